# Initial kernel scaffold; baseline (speedup 1.0000x reference)
import jax
import jax.numpy as jnp
from jax.experimental import pallas as pl


def kernel(xyz, new_xyz, features):
    raise NotImplementedError("write your pallas kernel here")


# speedup vs baseline: 5.5954x; 5.5954x over previous
import functools

import jax
import jax.numpy as jnp
from jax import lax
from jax.experimental import pallas as pl
from jax.experimental.pallas import tpu as pltpu
from jax.experimental.pallas import tpu_sc as plsc

N_SRC = 16384
N_QRY = 4096
C_FEAT = 128
K = 3

QB = 128

NC, NS = 2, 16
NW = NC * NS
BPW = N_QRY // NW


def _nn3_body(q_ref, xt_ref, idx_ref, w_ref):
    q = q_ref[...]
    qx, qy, qz = q[:, 0:1], q[:, 1:2], q[:, 2:3]
    xx = xt_ref[0:1, :]
    yy = xt_ref[1:2, :]
    zz = xt_ref[2:3, :]
    dx = qx - xx
    dy = qy - yy
    dz = qz - zz
    d2 = dx * dx + dy * dy + dz * dz

    iota = lax.broadcasted_iota(jnp.int32, (QB, N_SRC), 1)
    inf = jnp.float32(jnp.inf)
    big_i = jnp.int32(N_SRC)

    def amin(d):
        m = jnp.min(d, axis=1, keepdims=True)
        i = jnp.min(jnp.where(d == m, iota, big_i), axis=1, keepdims=True)
        return m, i

    m1, i1 = amin(d2)
    d2 = jnp.where(iota == i1, inf, d2)
    m2, i2 = amin(d2)
    d2 = jnp.where(iota == i2, inf, d2)
    m3, i3 = amin(d2)

    d1 = jnp.sqrt(jnp.maximum(m1, 1e-12))
    dd2 = jnp.sqrt(jnp.maximum(m2, 1e-12))
    dd3 = jnp.sqrt(jnp.maximum(m3, 1e-12))
    r1 = 1.0 / (d1 + 1e-8)
    r2 = 1.0 / (dd2 + 1e-8)
    r3 = 1.0 / (dd3 + 1e-8)
    norm = r1 + r2 + r3

    idx_ref[...] = jnp.concatenate([i1, i2, i3], axis=1)
    w_ref[...] = jnp.concatenate([r1 / norm, r2 / norm, r3 / norm], axis=1)


def _nn3(new_xyz, xt):
    return pl.pallas_call(
        _nn3_body,
        grid=(N_QRY // QB,),
        in_specs=[
            pl.BlockSpec((QB, 3), lambda i: (i, 0)),
            pl.BlockSpec((3, N_SRC), lambda i: (0, 0)),
        ],
        out_specs=[
            pl.BlockSpec((QB, 3), lambda i: (i, 0)),
            pl.BlockSpec((QB, 3), lambda i: (i, 0)),
        ],
        out_shape=[
            jax.ShapeDtypeStruct((N_QRY, 3), jnp.int32),
            jax.ShapeDtypeStruct((N_QRY, 3), jnp.float32),
        ],
    )(new_xyz, xt)


@functools.cache
def _sc_gather_fn():
    mesh = plsc.VectorSubcoreMesh(core_axis_name="c", subcore_axis_name="s")

    @functools.partial(
        pl.kernel,
        mesh=mesh,
        out_type=jax.ShapeDtypeStruct((K, N_QRY, C_FEAT), jnp.float32),
        scratch_types=[
            pltpu.VMEM((K, BPW), jnp.int32),
            pltpu.VMEM((K, BPW, C_FEAT), jnp.float32),
            pltpu.SemaphoreType.DMA,
        ],
    )
    def _sc_gather(idx_hbm, feat_hbm, out_hbm, idx_v, rows_v, sem):
        wid = lax.axis_index("s") * NC + lax.axis_index("c")
        base = wid * BPW
        pltpu.sync_copy(idx_hbm.at[:, pl.ds(base, BPW)], idx_v)
        for k in range(K):
            pltpu.async_copy(feat_hbm.at[idx_v.at[k]], rows_v.at[k], sem)
        for k in range(K):
            pltpu.make_async_copy(feat_hbm.at[idx_v.at[k]], rows_v.at[k],
                                  sem).wait()
        pltpu.sync_copy(rows_v, out_hbm.at[:, pl.ds(base, BPW), :])

    return _sc_gather


def _combine_body(g_ref, w_ref, o_ref):
    w = w_ref[...]
    o_ref[...] = (g_ref[0] * w[:, 0:1]
                  + g_ref[1] * w[:, 1:2]
                  + g_ref[2] * w[:, 2:3])


def _combine(g, w):
    return pl.pallas_call(
        _combine_body,
        grid=(N_QRY // QB,),
        in_specs=[
            pl.BlockSpec((K, QB, C_FEAT), lambda i: (0, i, 0)),
            pl.BlockSpec((QB, 3), lambda i: (i, 0)),
        ],
        out_specs=pl.BlockSpec((QB, C_FEAT), lambda i: (i, 0)),
        out_shape=jax.ShapeDtypeStruct((N_QRY, C_FEAT), jnp.float32),
    )(g, w)


def kernel(xyz, new_xyz, features):
    xt = xyz.T
    idx, w = _nn3(new_xyz, xt)
    g = _sc_gather_fn()(idx.T, features)
    return _combine(g, w)
